# baseline (speedup 1.0000x reference)
import functools

import jax
import jax.numpy as jnp
from jax import lax
from jax.experimental import pallas as pl
from jax.experimental.pallas import tpu as pltpu
from jax.experimental.pallas import tpu_sc as plsc

_INFO = plsc.get_sparse_core_info()
_NC = _INFO.num_cores
_NS = _INFO.num_subcores
_NW = _NC * _NS

_CHUNK = 128


@functools.partial(jax.jit, static_argnums=(2, 3))
def _sc_gather(idx3, table, n_total, d_model):
    n_per_w = n_total // _NW
    n_chunks = n_per_w // _CHUNK
    n_pairs = n_chunks // 2
    mesh = plsc.VectorSubcoreMesh(core_axis_name="c", subcore_axis_name="s")

    @functools.partial(
        pl.kernel,
        mesh=mesh,
        out_type=jax.ShapeDtypeStruct((n_total, d_model), jnp.float32),
        scratch_types=[
            pltpu.VMEM((n_chunks, _CHUNK), jnp.int32),
            pltpu.VMEM((4, _CHUNK, d_model), jnp.float32),
            pltpu.SemaphoreType.DMA,
            pltpu.SemaphoreType.DMA,
            pltpu.SemaphoreType.DMA,
            pltpu.SemaphoreType.DMA,
            pltpu.SemaphoreType.DMA,
            pltpu.SemaphoreType.DMA,
            pltpu.SemaphoreType.DMA,
            pltpu.SemaphoreType.DMA,
        ],
    )
    def k(table_hbm, idx_hbm, out_hbm, idx_v, rows_v,
          g0, g1, g2, g3, w0, w1, w2, w3):
        wid = lax.axis_index("s") * _NC + lax.axis_index("c")
        base = wid * n_per_w
        sem_g = (g0, g1, g2, g3)
        sem_w = (w0, w1, w2, w3)

        pltpu.sync_copy(idx_hbm.at[wid], idx_v)

        def start_gather(j, b):
            jj = jnp.minimum(j, n_chunks - 1)
            pltpu.async_copy(table_hbm.at[idx_v.at[jj]], rows_v.at[b], sem_g[b])

        def wait_gather(b):
            pltpu.make_async_copy(
                table_hbm.at[idx_v.at[0]], rows_v.at[b], sem_g[b]).wait()

        def start_wb(j, b):
            pltpu.async_copy(
                rows_v.at[b], out_hbm.at[pl.ds(base + j * _CHUNK, _CHUNK)],
                sem_w[b])

        def wait_wb(b):
            pltpu.make_async_copy(
                rows_v.at[b], out_hbm.at[pl.ds(base, _CHUNK)], sem_w[b]).wait()

        def block(i, first):
            j = 4 * i
            if not first:
                wait_wb(2)
            start_gather(j + 2, 2)
            if not first:
                wait_wb(3)
            start_gather(j + 3, 3)
            wait_gather(0)
            start_wb(j + 0, 0)
            wait_gather(1)
            start_wb(j + 1, 1)
            wait_wb(0)
            start_gather(j + 4, 0)
            wait_wb(1)
            start_gather(j + 5, 1)
            wait_gather(2)
            start_wb(j + 2, 2)
            wait_gather(3)
            start_wb(j + 3, 3)

        start_gather(0, 0)
        start_gather(1, 1)
        block(0, first=True)

        def body(i, carry):
            block(i, first=False)
            return carry

        lax.fori_loop(1, n_chunks // 4, body, 0)

        wait_wb(2)
        wait_wb(3)
        wait_gather(0)
        wait_gather(1)

    return k(table, idx3)


def kernel(x, table):
    n_total = x.shape[0] * x.shape[1]
    d_model = table.shape[1]
    n_per_w = n_total // _NW
    idx3 = x.reshape(_NW, n_per_w // _CHUNK, _CHUNK).astype(jnp.int32)
    out = _sc_gather(idx3, table, n_total, d_model)
    return out.reshape(x.shape[0], x.shape[1], d_model)


# speedup vs baseline: 1.0067x; 1.0067x over previous
import functools

import jax
import jax.numpy as jnp
from jax import lax
from jax.experimental import pallas as pl
from jax.experimental.pallas import tpu as pltpu
from jax.experimental.pallas import tpu_sc as plsc

_INFO = plsc.get_sparse_core_info()
_NC = _INFO.num_cores
_NS = _INFO.num_subcores
_NW = _NC * _NS

_CHUNK = 128


@functools.partial(jax.jit, static_argnums=(2, 3))
def _sc_gather(idx3, table, n_total, d_model):
    n_per_w = n_total // _NW
    n_chunks = n_per_w // _CHUNK
    mesh = plsc.VectorSubcoreMesh(core_axis_name="c", subcore_axis_name="s")

    @functools.partial(
        pl.kernel,
        mesh=mesh,
        out_type=jax.ShapeDtypeStruct((n_total, d_model), jnp.float32),
        scratch_types=[
            pltpu.VMEM((n_chunks, _CHUNK), jnp.int32),
            pltpu.VMEM((_CHUNK, d_model), jnp.float32),
            pltpu.VMEM((_CHUNK, d_model), jnp.float32),
            pltpu.SemaphoreType.DMA,
            pltpu.SemaphoreType.DMA,
        ],
    )
    def k(table_hbm, idx_hbm, out_hbm, idx_v, rows0, rows1, sem0, sem1):
        wid = lax.axis_index("s") * _NC + lax.axis_index("c")
        base = wid * n_per_w

        pltpu.sync_copy(idx_hbm.at[wid], idx_v)

        pltpu.async_copy(table_hbm.at[idx_v.at[0]], rows0, sem0)

        def body(i, carry):
            j0 = 2 * i
            j1 = j0 + 1
            pltpu.async_copy(table_hbm.at[idx_v.at[j1]], rows1, sem1)
            pltpu.make_async_copy(table_hbm.at[idx_v.at[j0]], rows0, sem0).wait()
            pltpu.sync_copy(rows0, out_hbm.at[pl.ds(base + j0 * _CHUNK, _CHUNK)])
            j2 = jnp.minimum(j0 + 2, n_chunks - 1)
            pltpu.async_copy(table_hbm.at[idx_v.at[j2]], rows0, sem0)
            pltpu.make_async_copy(table_hbm.at[idx_v.at[j1]], rows1, sem1).wait()
            pltpu.sync_copy(rows1, out_hbm.at[pl.ds(base + j1 * _CHUNK, _CHUNK)])
            return carry

        lax.fori_loop(0, n_chunks // 2, body, 0)
        pltpu.make_async_copy(table_hbm.at[idx_v.at[0]], rows0, sem0).wait()

    return k(table, idx3)


def kernel(x, table):
    n_total = x.shape[0] * x.shape[1]
    d_model = table.shape[1]
    n_per_w = n_total // _NW
    idx3 = x.reshape(_NW, n_per_w // _CHUNK, _CHUNK).astype(jnp.int32)
    out = _sc_gather(idx3, table, n_total, d_model)
    return out.reshape(x.shape[0], x.shape[1], d_model)
